# baseline (speedup 1.0000x reference)
import numpy as np
import jax
import jax.numpy as jnp
from jax.experimental import pallas as pl
from jax.experimental.pallas import tpu as pltpu

IMAGE_SIZE = 224
MAX_RADIUS = 112
BN_EPS = 1e-5
NPIX = IMAGE_SIZE * IMAGE_SIZE
F32 = jnp.float32
BF = jnp.bfloat16

_yy, _xx = np.meshgrid(np.arange(IMAGE_SIZE), np.arange(IMAGE_SIZE), indexing="ij")
_r = np.sqrt((_xx - IMAGE_SIZE // 2) ** 2 + (_yy - IMAGE_SIZE // 2) ** 2)
_rint = np.round(_r).astype(np.int32)
_RIDS = np.where(_rint < MAX_RADIUS, _rint, MAX_RADIUS).reshape(-1)
_OH = np.zeros((NPIX, 128), np.float32)
_OH[np.arange(NPIX), _RIDS] = 1.0
_COUNTS = _OH.sum(0)[:MAX_RADIUS]
_CNTR = np.zeros((1, 128), np.float32)
_CNTR[0, :MAX_RADIUS] = 1.0 / (_COUNTS + 1e-8)

_ks = (np.arange(IMAGE_SIZE) + IMAGE_SIZE // 2) % IMAGE_SIZE
_ang = -2.0 * np.pi * np.outer(_ks, np.arange(IMAGE_SIZE)) / IMAGE_SIZE
_FSR = np.cos(_ang).astype(np.float32)
_FSI = np.sin(_ang).astype(np.float32)
_FSRT = np.ascontiguousarray(_FSR.T)
_FSIT = np.ascontiguousarray(_FSI.T)

_EW = np.zeros((224, 840), np.float32)
for _xb in range(28):
    for _dy in range(3):
        for _dd in range(10):
            _x = 8 * _xb + _dd - 1
            if 0 <= _x < 224:
                _EW[_x, 30 * _xb + 10 * _dy + _dd] = 1.0


def _silu(v):
    return v * jax.nn.sigmoid(v)


def _hilo(v):
    h = v.astype(BF)
    return h, (v - h.astype(F32)).astype(BF)


def _fft_kernel(x_ref, fr_ref, fi_ref, frt_ref, fit_ref, ew_ref,
                o_ref, o2_ref):
    nb = x_ref.shape[0]
    x = x_ref[...]
    gray = 0.299 * x[:, 0] + 0.587 * x[:, 1] + 0.114 * x[:, 2]
    g2 = gray.reshape(nb * 224, 224)
    hp = jax.lax.Precision.HIGHEST
    ur = jnp.dot(g2, frt_ref[...], preferred_element_type=F32, precision=hp)
    ui = jnp.dot(g2, fit_ref[...], preferred_element_type=F32, precision=hp)
    fr, fi = fr_ref[...], fi_ref[...]
    ew = ew_ref[...]
    for s in range(nb):
        urs = ur[s * 224:(s + 1) * 224]
        uis = ui[s * 224:(s + 1) * 224]
        vr = (jnp.dot(fr, urs, preferred_element_type=F32, precision=hp)
              - jnp.dot(fi, uis, preferred_element_type=F32, precision=hp))
        vi = (jnp.dot(fr, uis, preferred_element_type=F32, precision=hp)
              + jnp.dot(fi, urs, preferred_element_type=F32, precision=hp))
        lm = jnp.log1p(jnp.sqrt(vr * vr + vi * vi))
        m = jnp.sum(lm, keepdims=True) * (1.0 / NPIX)
        d = lm - m
        var = jnp.sum(d * d, keepdims=True) * (1.0 / (NPIX - 1))
        d = d / (jnp.sqrt(var) + 1e-8)
        o_ref[s] = d
        dh, dl = _hilo(d)
        o2_ref[s, 0:1, :] = jnp.zeros((1, 840), F32)
        o2_ref[s, 225:226, :] = jnp.zeros((1, 840), F32)
        o2_ref[s, 1:225, :] = (jnp.dot(dh, ew, preferred_element_type=F32)
                               + jnp.dot(dl, ew, preferred_element_type=F32))


_KCHUNKS = 8
_KC = NPIX // _KCHUNKS


def _hist_kernel(spec_ref, oh_ref, cntr_ref, w1_ref, b1_ref, w2_ref, b2_ref,
                 o_ref, acc_ref):
    k = pl.program_id(0)

    @pl.when(k == 0)
    def _():
        acc_ref[...] = jnp.zeros_like(acc_ref)

    acc_ref[...] += jnp.dot(spec_ref[...], oh_ref[...],
                            preferred_element_type=F32)

    @pl.when(k == _KCHUNKS - 1)
    def _():
        b = acc_ref.shape[0]
        radial = acc_ref[...] * cntr_ref[...]
        lane = jax.lax.broadcasted_iota(jnp.int32, (b, 128), 1)
        mask = lane < MAX_RADIUS
        mean = jnp.sum(radial, axis=1, keepdims=True) * (1.0 / MAX_RADIUS)
        d = jnp.where(mask, radial - mean, 0.0)
        var = jnp.sum(d * d, axis=1, keepdims=True) * (1.0 / (MAX_RADIUS - 1))
        rn = d / (jnp.sqrt(var) + 1e-8)
        g = jnp.dot(rn, w1_ref[...], preferred_element_type=F32) + b1_ref[...]
        g = _silu(g)
        h = jnp.dot(g, w2_ref[...], preferred_element_type=F32) + b2_ref[...]
        o_ref[...] = _silu(h)


def _cnn_kernel(s_ref, w1_ref, b1_ref, w2_ref, b2_ref, w3_ref, b3_ref,
                o_ref, p1, h1w, p2, h2w, p3):
    @pl.when(pl.program_id(0) == 0)
    def _():
        p1[:, 0:28, 30:32] = jnp.zeros((224, 28, 2), F32)
        p1[:, 28:32, :] = jnp.zeros((224, 4, 32), F32)
        p2[:, 28:32, :] = jnp.zeros((112, 4, 768), F32)
        p3[:, 28:32, :] = jnp.zeros((56, 4, 768), F32)
        h1w[0:1] = jnp.zeros((1, 28, 256), F32)
        h1w[113:114] = jnp.zeros((1, 28, 256), F32)
        h1w[:, :, 160:224] = jnp.zeros((114, 28, 64), F32)
        h2w[0:1] = jnp.zeros((1, 28, 256), F32)
        h2w[57:58] = jnp.zeros((1, 28, 256), F32)

    s = s_ref[0]
    p1[:, 0:28, 0:10] = s[0:224, :, 0:10]
    p1[:, 0:28, 10:20] = s[1:225, :, 10:20]
    p1[:, 0:28, 20:30] = s[2:226, :, 20:30]
    w1v, b1v = w1_ref[...], b1_ref[...]
    for c in range(14):
        pc = p1[16 * c:16 * (c + 1)]
        zc = jnp.dot(pc.reshape(512, 32), w1v,
                     preferred_element_type=F32) + b1v
        v = _silu(zc).reshape(8, 2, 32, 256)
        m = jnp.maximum(v[:, 0], v[:, 1])
        m2 = jnp.maximum(m[:, :, 0:128], m[:, :, 128:256])
        h1w[1 + 8 * c:9 + 8 * c, :, 0:128] = m2[:, 0:28, :]
        h1w[1 + 8 * c:9 + 8 * c, :, 128:160] = jnp.concatenate(
            [m2[:, 1:28, 0:32], jnp.zeros((8, 1, 32), F32)], axis=1)
        h1w[1 + 8 * c:9 + 8 * c, :, 224:256] = jnp.concatenate(
            [jnp.zeros((8, 1, 32), F32), m2[:, 0:27, 96:128]], axis=1)
    for dy in range(3):
        p2[:, 0:28, 256 * dy:256 * (dy + 1)] = h1w[dy:dy + 112]
    w2v, b2v = w2_ref[...], b2_ref[...]
    for c in range(7):
        pc = p2[16 * c:16 * (c + 1)]
        zc = jnp.dot(pc.reshape(512, 768), w2v,
                     preferred_element_type=F32) + b2v
        v = _silu(zc).reshape(8, 2, 32, 256)
        m = jnp.maximum(v[:, 0], v[:, 1])
        m2 = jnp.maximum(m[:, :, 0:128], m[:, :, 128:256])
        h2w[1 + 8 * c:9 + 8 * c, :, 0:128] = m2[:, 0:28, :]
        h2w[1 + 8 * c:9 + 8 * c, :, 128:192] = jnp.concatenate(
            [m2[:, 1:28, 0:64], jnp.zeros((8, 1, 64), F32)], axis=1)
        h2w[1 + 8 * c:9 + 8 * c, :, 192:256] = jnp.concatenate(
            [jnp.zeros((8, 1, 64), F32), m2[:, 0:27, 64:128]], axis=1)
    for dy in range(3):
        p3[:, 0:28, 256 * dy:256 * (dy + 1)] = h2w[dy:dy + 56]
    w3v, b3v = w3_ref[...], b3_ref[...]
    tot = jnp.zeros((1, 256), F32)
    for c in range(4):
        pc = p3[14 * c:14 * (c + 1)]
        zc = jnp.dot(pc.reshape(448, 768), w3v,
                     preferred_element_type=F32) + b3v
        sv = _silu(zc).reshape(14, 32, 256)[:, 0:28, :]
        tot = tot + jnp.sum(jnp.sum(sv, axis=0), axis=0, keepdims=True)
    o_ref[0] = (tot[:, 0:128] + tot[:, 128:256]) * (1.0 / 3136.0)


def _head_kernel(c_ref, w1_ref, b1_ref, w2_ref, b2_ref, o_ref):
    z = jnp.dot(c_ref[...], w1_ref[...], preferred_element_type=F32) + b1_ref[...]
    z = _silu(z)
    o_ref[...] = jnp.dot(z, w2_ref[...], preferred_element_type=F32) + b2_ref[...]


def _bn_fold(bn):
    s = bn["g"] / jnp.sqrt(bn["v"] + BN_EPS)
    t = bn["b"] - bn["m"] * s
    return s, t


_ROFF2 = [224, 0, 32, 64, 96, 128]
_ROFF3 = [192, 0, 64, 128]


def kernel(x, params):
    p = params
    B = x.shape[0]

    s1, t1 = _bn_fold(p["bn1"])
    s2, t2 = _bn_fold(p["bn2"])
    s3, t3 = _bn_fold(p["bn3"])

    w1s = p["c1_w"][:, 0] * s1[:, None, None]
    w1b = jnp.zeros((32, 256), F32)
    for dy in range(3):
        for kx in range(3):
            vec = w1s[:, dy, kx]
            for j in range(8):
                c0 = (j % 2) * 128 + (j // 2) * 32
                w1b = w1b.at[dy * 10 + j + kx, c0:c0 + 32].set(vec)

    w2s = p["c2_w"] * s2[:, None, None, None]
    w2b = jnp.zeros((768, 256), F32)
    for dy in range(3):
        for kx in range(3):
            blk = w2s[:, :, dy, kx].T
            for j2 in range(4):
                r0 = dy * 256 + _ROFF2[j2 + kx]
                c0 = (j2 % 2) * 128 + (j2 // 2) * 64
                w2b = w2b.at[r0:r0 + 32, c0:c0 + 64].set(blk)

    w3s = p["c3_w"] * s3[:, None, None, None]
    w3b = jnp.zeros((768, 256), F32)
    for dy in range(3):
        for kx in range(3):
            blk = w3s[:, :, dy, kx].T
            for j3 in range(2):
                r0 = dy * 256 + _ROFF3[j3 + kx]
                w3b = w3b.at[r0:r0 + 64, j3 * 128:(j3 + 1) * 128].set(blk)

    b1b = jnp.tile(p["c1_b"] * s1 + t1, 8)[None, :]
    b2b = jnp.tile(p["c2_b"] * s2 + t2, 4)[None, :]
    b3b = jnp.tile(p["c3_b"] * s3 + t3, 2)[None, :]

    s1d, t1d = _bn_fold(p["bn1d"])
    w1e = jnp.zeros((128, 64), F32).at[:MAX_RADIUS, :].set(
        p["l1_w"].T * s1d[None, :])
    b1e = (p["l1_b"] * s1d + t1d)[None, :]
    w2e = p["l2_w"].T
    b2e = p["l2_b"][None, :]

    wf1 = p["f1_w"].T
    bf1 = p["f1_b"][None, :]
    wf2 = jnp.zeros((64, 128), F32).at[:, 0].set(p["f2_w"][0])
    bf2 = jnp.zeros((1, 128), F32).at[0, 0].set(p["f2_b"][0])

    fsr, fsi = jnp.asarray(_FSR), jnp.asarray(_FSI)
    fsrt, fsit = jnp.asarray(_FSRT), jnp.asarray(_FSIT)
    ewb = jnp.asarray(_EW).astype(BF)

    nb = 8
    c224 = pl.BlockSpec((224, 224), lambda i: (0, 0))
    spec, sdec2 = pl.pallas_call(
        _fft_kernel,
        out_shape=(jax.ShapeDtypeStruct((B, 224, 224), F32),
                   jax.ShapeDtypeStruct((B, 226, 840), F32)),
        grid=(B // nb,),
        in_specs=[pl.BlockSpec((nb, 3, 224, 224), lambda i: (i, 0, 0, 0)),
                  c224, c224, c224, c224,
                  pl.BlockSpec((224, 840), lambda i: (0, 0))],
        out_specs=[pl.BlockSpec((nb, 224, 224), lambda i: (i, 0, 0)),
                   pl.BlockSpec((nb, 226, 840), lambda i: (i, 0, 0))],
        compiler_params=pltpu.CompilerParams(
            dimension_semantics=("arbitrary",),
            vmem_limit_bytes=52 * 1024 * 1024,
        ),
        name="fft_spectrum",
    )(x, fsr, fsi, fsrt, fsit, ewb)

    spec_flat = spec.reshape(B, NPIX)
    feat1d = pl.pallas_call(
        _hist_kernel,
        out_shape=jax.ShapeDtypeStruct((B, 64), F32),
        grid=(_KCHUNKS,),
        in_specs=[
            pl.BlockSpec((B, _KC), lambda k: (0, k)),
            pl.BlockSpec((_KC, 128), lambda k: (k, 0)),
            pl.BlockSpec((1, 128), lambda k: (0, 0)),
            pl.BlockSpec((128, 64), lambda k: (0, 0)),
            pl.BlockSpec((1, 64), lambda k: (0, 0)),
            pl.BlockSpec((64, 64), lambda k: (0, 0)),
            pl.BlockSpec((1, 64), lambda k: (0, 0)),
        ],
        out_specs=pl.BlockSpec((B, 64), lambda k: (0, 0)),
        scratch_shapes=[pltpu.VMEM((B, 128), F32)],
        compiler_params=pltpu.CompilerParams(
            dimension_semantics=("arbitrary",),
            vmem_limit_bytes=48 * 1024 * 1024,
        ),
        name="radial_hist_mlp",
    )(spec_flat, jnp.asarray(_OH), jnp.asarray(_CNTR), w1e, b1e, w2e, b2e)

    sdec = sdec2.reshape(B, 226, 28, 30)
    feat2d = pl.pallas_call(
        _cnn_kernel,
        out_shape=jax.ShapeDtypeStruct((B, 1, 128), F32),
        grid=(B,),
        in_specs=[
            pl.BlockSpec((1, 226, 28, 30), lambda i: (i, 0, 0, 0)),
            pl.BlockSpec((32, 256), lambda i: (0, 0)),
            pl.BlockSpec((1, 256), lambda i: (0, 0)),
            pl.BlockSpec((768, 256), lambda i: (0, 0)),
            pl.BlockSpec((1, 256), lambda i: (0, 0)),
            pl.BlockSpec((768, 256), lambda i: (0, 0)),
            pl.BlockSpec((1, 256), lambda i: (0, 0)),
        ],
        out_specs=pl.BlockSpec((1, 1, 128), lambda i: (i, 0, 0)),
        scratch_shapes=[
            pltpu.VMEM((224, 32, 32), F32),
            pltpu.VMEM((114, 28, 256), F32),
            pltpu.VMEM((112, 32, 768), F32),
            pltpu.VMEM((58, 28, 256), F32),
            pltpu.VMEM((56, 32, 768), F32),
        ],
        compiler_params=pltpu.CompilerParams(
            dimension_semantics=("arbitrary",),
            vmem_limit_bytes=56 * 1024 * 1024,
        ),
        name="cnn_tower",
    )(sdec, w1b, b1b, w2b, b2b, w3b, b3b)

    combined = jnp.concatenate([feat2d[:, 0, :], feat1d], axis=1)
    out = pl.pallas_call(
        _head_kernel,
        out_shape=jax.ShapeDtypeStruct((B, 128), F32),
        name="head_mlp",
    )(combined, wf1, bf1, wf2, bf2)
    return out[:, :1]


# speedup vs baseline: 1.0244x; 1.0244x over previous
import numpy as np
import jax
import jax.numpy as jnp
from jax.experimental import pallas as pl
from jax.experimental.pallas import tpu as pltpu

IMAGE_SIZE = 224
MAX_RADIUS = 112
BN_EPS = 1e-5
NPIX = IMAGE_SIZE * IMAGE_SIZE
F32 = jnp.float32
BF = jnp.bfloat16

_yy, _xx = np.meshgrid(np.arange(IMAGE_SIZE), np.arange(IMAGE_SIZE), indexing="ij")
_r = np.sqrt((_xx - IMAGE_SIZE // 2) ** 2 + (_yy - IMAGE_SIZE // 2) ** 2)
_rint = np.round(_r).astype(np.int32)
_RIDS = np.where(_rint < MAX_RADIUS, _rint, MAX_RADIUS).reshape(-1)
_OH = np.zeros((NPIX, 128), np.float32)
_OH[np.arange(NPIX), _RIDS] = 1.0
_COUNTS = _OH.sum(0)[:MAX_RADIUS]
_CNTR = np.zeros((1, 128), np.float32)
_CNTR[0, :MAX_RADIUS] = 1.0 / (_COUNTS + 1e-8)

_ks = (np.arange(IMAGE_SIZE) + IMAGE_SIZE // 2) % IMAGE_SIZE
_ang = -2.0 * np.pi * np.outer(_ks, np.arange(IMAGE_SIZE)) / IMAGE_SIZE
_FSR = np.cos(_ang).astype(np.float32)
_FSI = np.sin(_ang).astype(np.float32)
_FSRT = np.ascontiguousarray(_FSR.T)
_FSIT = np.ascontiguousarray(_FSI.T)

_EW = np.zeros((224, 840), np.float32)
for _xb in range(28):
    for _dy in range(3):
        for _dd in range(10):
            _x = 8 * _xb + _dd - 1
            if 0 <= _x < 224:
                _EW[_x, 30 * _xb + 10 * _dy + _dd] = 1.0


def _silu(v):
    return v * jax.nn.sigmoid(v)


def _hilo(v):
    h = v.astype(BF)
    return h, (v - h.astype(F32)).astype(BF)


def _fft_kernel(x_ref, fr_ref, fi_ref, frt_ref, fit_ref, ew_ref,
                o_ref, o2_ref):
    nb = x_ref.shape[0]
    x = x_ref[...]
    gray = 0.299 * x[:, 0] + 0.587 * x[:, 1] + 0.114 * x[:, 2]
    g2 = gray.reshape(nb * 224, 224)
    hp = jax.lax.Precision.HIGHEST
    ur = jnp.dot(g2, frt_ref[...], preferred_element_type=F32, precision=hp)
    ui = jnp.dot(g2, fit_ref[...], preferred_element_type=F32, precision=hp)
    fr, fi = fr_ref[...], fi_ref[...]
    ew = ew_ref[...]
    for s in range(nb):
        urs = ur[s * 224:(s + 1) * 224]
        uis = ui[s * 224:(s + 1) * 224]
        vr = (jnp.dot(fr, urs, preferred_element_type=F32, precision=hp)
              - jnp.dot(fi, uis, preferred_element_type=F32, precision=hp))
        vi = (jnp.dot(fr, uis, preferred_element_type=F32, precision=hp)
              + jnp.dot(fi, urs, preferred_element_type=F32, precision=hp))
        lm = jnp.log1p(jnp.sqrt(vr * vr + vi * vi))
        m = jnp.sum(lm, keepdims=True) * (1.0 / NPIX)
        d = lm - m
        var = jnp.sum(d * d, keepdims=True) * (1.0 / (NPIX - 1))
        d = d / (jnp.sqrt(var) + 1e-8)
        o_ref[s] = d
        dh, dl = _hilo(d)
        o2_ref[s, 0:1, :] = jnp.zeros((1, 840), F32)
        o2_ref[s, 225:226, :] = jnp.zeros((1, 840), F32)
        o2_ref[s, 1:225, :] = (jnp.dot(dh, ew, preferred_element_type=F32)
                               + jnp.dot(dl, ew, preferred_element_type=F32))


_KCHUNKS = 7
_KROWS = 224 // _KCHUNKS


def _hist_kernel(spec_ref, oh_ref, cntr_ref, w1_ref, b1_ref, w2_ref, b2_ref,
                 o_ref, acc_ref):
    k = pl.program_id(0)

    @pl.when(k == 0)
    def _():
        acc_ref[...] = jnp.zeros_like(acc_ref)

    sp = spec_ref[...]
    tot = jnp.dot(sp[:, 0, :], oh_ref[0], preferred_element_type=F32)
    for y in range(1, _KROWS):
        tot = tot + jnp.dot(sp[:, y, :], oh_ref[y], preferred_element_type=F32)
    acc_ref[...] += tot

    @pl.when(k == _KCHUNKS - 1)
    def _():
        b = acc_ref.shape[0]
        radial = acc_ref[...] * cntr_ref[...]
        lane = jax.lax.broadcasted_iota(jnp.int32, (b, 128), 1)
        mask = lane < MAX_RADIUS
        mean = jnp.sum(radial, axis=1, keepdims=True) * (1.0 / MAX_RADIUS)
        d = jnp.where(mask, radial - mean, 0.0)
        var = jnp.sum(d * d, axis=1, keepdims=True) * (1.0 / (MAX_RADIUS - 1))
        rn = d / (jnp.sqrt(var) + 1e-8)
        g = jnp.dot(rn, w1_ref[...], preferred_element_type=F32) + b1_ref[...]
        g = _silu(g)
        h = jnp.dot(g, w2_ref[...], preferred_element_type=F32) + b2_ref[...]
        o_ref[...] = _silu(h)


def _cnn_kernel(s_ref, w1_ref, b1_ref, w2_ref, b2_ref, w3_ref, b3_ref,
                o_ref, p1, h1w, p2, h2w, p3):
    @pl.when(pl.program_id(0) == 0)
    def _():
        p1[:, 0:28, 30:32] = jnp.zeros((224, 28, 2), F32)
        p1[:, 28:32, :] = jnp.zeros((224, 4, 32), F32)
        p2[:, 28:32, :] = jnp.zeros((112, 4, 768), F32)
        p3[:, 28:32, :] = jnp.zeros((56, 4, 768), F32)
        h1w[0:1] = jnp.zeros((1, 28, 256), F32)
        h1w[113:114] = jnp.zeros((1, 28, 256), F32)
        h1w[:, :, 160:224] = jnp.zeros((114, 28, 64), F32)
        h2w[0:1] = jnp.zeros((1, 28, 256), F32)
        h2w[57:58] = jnp.zeros((1, 28, 256), F32)

    s = s_ref[0]
    p1[:, 0:28, 0:10] = s[0:224, :, 0:10]
    p1[:, 0:28, 10:20] = s[1:225, :, 10:20]
    p1[:, 0:28, 20:30] = s[2:226, :, 20:30]
    w1v, b1v = w1_ref[...], b1_ref[...]
    for c in range(14):
        pc = p1[16 * c:16 * (c + 1)]
        zc = jnp.dot(pc.reshape(512, 32), w1v,
                     preferred_element_type=F32) + b1v
        v = _silu(zc).reshape(8, 2, 32, 256)
        m = jnp.maximum(v[:, 0], v[:, 1])
        m2 = jnp.maximum(m[:, :, 0:128], m[:, :, 128:256])
        h1w[1 + 8 * c:9 + 8 * c, :, 0:128] = m2[:, 0:28, :]
        h1w[1 + 8 * c:9 + 8 * c, :, 128:160] = jnp.concatenate(
            [m2[:, 1:28, 0:32], jnp.zeros((8, 1, 32), F32)], axis=1)
        h1w[1 + 8 * c:9 + 8 * c, :, 224:256] = jnp.concatenate(
            [jnp.zeros((8, 1, 32), F32), m2[:, 0:27, 96:128]], axis=1)
    for dy in range(3):
        p2[:, 0:28, 256 * dy:256 * (dy + 1)] = h1w[dy:dy + 112]
    w2v, b2v = w2_ref[...], b2_ref[...]
    for c in range(7):
        pc = p2[16 * c:16 * (c + 1)]
        zc = jnp.dot(pc.reshape(512, 768), w2v,
                     preferred_element_type=F32) + b2v
        v = _silu(zc).reshape(8, 2, 32, 256)
        m = jnp.maximum(v[:, 0], v[:, 1])
        m2 = jnp.maximum(m[:, :, 0:128], m[:, :, 128:256])
        h2w[1 + 8 * c:9 + 8 * c, :, 0:128] = m2[:, 0:28, :]
        h2w[1 + 8 * c:9 + 8 * c, :, 128:192] = jnp.concatenate(
            [m2[:, 1:28, 0:64], jnp.zeros((8, 1, 64), F32)], axis=1)
        h2w[1 + 8 * c:9 + 8 * c, :, 192:256] = jnp.concatenate(
            [jnp.zeros((8, 1, 64), F32), m2[:, 0:27, 64:128]], axis=1)
    for dy in range(3):
        p3[:, 0:28, 256 * dy:256 * (dy + 1)] = h2w[dy:dy + 56]
    w3v, b3v = w3_ref[...], b3_ref[...]
    tot = jnp.zeros((1, 256), F32)
    for c in range(4):
        pc = p3[14 * c:14 * (c + 1)]
        zc = jnp.dot(pc.reshape(448, 768), w3v,
                     preferred_element_type=F32) + b3v
        sv = _silu(zc).reshape(14, 32, 256)[:, 0:28, :]
        tot = tot + jnp.sum(jnp.sum(sv, axis=0), axis=0, keepdims=True)
    o_ref[0] = (tot[:, 0:128] + tot[:, 128:256]) * (1.0 / 3136.0)


def _head_kernel(c_ref, w1_ref, b1_ref, w2_ref, b2_ref, o_ref):
    z = jnp.dot(c_ref[...], w1_ref[...], preferred_element_type=F32) + b1_ref[...]
    z = _silu(z)
    o_ref[...] = jnp.dot(z, w2_ref[...], preferred_element_type=F32) + b2_ref[...]


def _bn_fold(bn):
    s = bn["g"] / jnp.sqrt(bn["v"] + BN_EPS)
    t = bn["b"] - bn["m"] * s
    return s, t


_ROFF2 = [224, 0, 32, 64, 96, 128]
_ROFF3 = [192, 0, 64, 128]


def kernel(x, params):
    p = params
    B = x.shape[0]

    s1, t1 = _bn_fold(p["bn1"])
    s2, t2 = _bn_fold(p["bn2"])
    s3, t3 = _bn_fold(p["bn3"])

    w1s = p["c1_w"][:, 0] * s1[:, None, None]
    w1b = jnp.zeros((32, 256), F32)
    for dy in range(3):
        for kx in range(3):
            vec = w1s[:, dy, kx]
            for j in range(8):
                c0 = (j % 2) * 128 + (j // 2) * 32
                w1b = w1b.at[dy * 10 + j + kx, c0:c0 + 32].set(vec)

    w2s = p["c2_w"] * s2[:, None, None, None]
    w2b = jnp.zeros((768, 256), F32)
    for dy in range(3):
        for kx in range(3):
            blk = w2s[:, :, dy, kx].T
            for j2 in range(4):
                r0 = dy * 256 + _ROFF2[j2 + kx]
                c0 = (j2 % 2) * 128 + (j2 // 2) * 64
                w2b = w2b.at[r0:r0 + 32, c0:c0 + 64].set(blk)

    w3s = p["c3_w"] * s3[:, None, None, None]
    w3b = jnp.zeros((768, 256), F32)
    for dy in range(3):
        for kx in range(3):
            blk = w3s[:, :, dy, kx].T
            for j3 in range(2):
                r0 = dy * 256 + _ROFF3[j3 + kx]
                w3b = w3b.at[r0:r0 + 64, j3 * 128:(j3 + 1) * 128].set(blk)

    b1b = jnp.tile(p["c1_b"] * s1 + t1, 8)[None, :]
    b2b = jnp.tile(p["c2_b"] * s2 + t2, 4)[None, :]
    b3b = jnp.tile(p["c3_b"] * s3 + t3, 2)[None, :]

    s1d, t1d = _bn_fold(p["bn1d"])
    w1e = jnp.zeros((128, 64), F32).at[:MAX_RADIUS, :].set(
        p["l1_w"].T * s1d[None, :])
    b1e = (p["l1_b"] * s1d + t1d)[None, :]
    w2e = p["l2_w"].T
    b2e = p["l2_b"][None, :]

    wf1 = p["f1_w"].T
    bf1 = p["f1_b"][None, :]
    wf2 = jnp.zeros((64, 128), F32).at[:, 0].set(p["f2_w"][0])
    bf2 = jnp.zeros((1, 128), F32).at[0, 0].set(p["f2_b"][0])

    fsr, fsi = jnp.asarray(_FSR), jnp.asarray(_FSI)
    fsrt, fsit = jnp.asarray(_FSRT), jnp.asarray(_FSIT)
    ewb = jnp.asarray(_EW).astype(BF)

    nb = 8
    c224 = pl.BlockSpec((224, 224), lambda i: (0, 0))
    spec, sdec2 = pl.pallas_call(
        _fft_kernel,
        out_shape=(jax.ShapeDtypeStruct((B, 224, 224), F32),
                   jax.ShapeDtypeStruct((B, 226, 840), F32)),
        grid=(B // nb,),
        in_specs=[pl.BlockSpec((nb, 3, 224, 224), lambda i: (i, 0, 0, 0)),
                  c224, c224, c224, c224,
                  pl.BlockSpec((224, 840), lambda i: (0, 0))],
        out_specs=[pl.BlockSpec((nb, 224, 224), lambda i: (i, 0, 0)),
                   pl.BlockSpec((nb, 226, 840), lambda i: (i, 0, 0))],
        compiler_params=pltpu.CompilerParams(
            dimension_semantics=("arbitrary",),
            vmem_limit_bytes=52 * 1024 * 1024,
        ),
        name="fft_spectrum",
    )(x, fsr, fsi, fsrt, fsit, ewb)

    feat1d = pl.pallas_call(
        _hist_kernel,
        out_shape=jax.ShapeDtypeStruct((B, 64), F32),
        grid=(_KCHUNKS,),
        in_specs=[
            pl.BlockSpec((B, _KROWS, 224), lambda k: (0, k, 0)),
            pl.BlockSpec((_KROWS, 224, 128), lambda k: (k, 0, 0)),
            pl.BlockSpec((1, 128), lambda k: (0, 0)),
            pl.BlockSpec((128, 64), lambda k: (0, 0)),
            pl.BlockSpec((1, 64), lambda k: (0, 0)),
            pl.BlockSpec((64, 64), lambda k: (0, 0)),
            pl.BlockSpec((1, 64), lambda k: (0, 0)),
        ],
        out_specs=pl.BlockSpec((B, 64), lambda k: (0, 0)),
        scratch_shapes=[pltpu.VMEM((B, 128), F32)],
        compiler_params=pltpu.CompilerParams(
            dimension_semantics=("arbitrary",),
            vmem_limit_bytes=48 * 1024 * 1024,
        ),
        name="radial_hist_mlp",
    )(spec, jnp.asarray(_OH).reshape(224, 224, 128),
      jnp.asarray(_CNTR), w1e, b1e, w2e, b2e)

    sdec = sdec2.reshape(B, 226, 28, 30)
    feat2d = pl.pallas_call(
        _cnn_kernel,
        out_shape=jax.ShapeDtypeStruct((B, 1, 128), F32),
        grid=(B,),
        in_specs=[
            pl.BlockSpec((1, 226, 28, 30), lambda i: (i, 0, 0, 0)),
            pl.BlockSpec((32, 256), lambda i: (0, 0)),
            pl.BlockSpec((1, 256), lambda i: (0, 0)),
            pl.BlockSpec((768, 256), lambda i: (0, 0)),
            pl.BlockSpec((1, 256), lambda i: (0, 0)),
            pl.BlockSpec((768, 256), lambda i: (0, 0)),
            pl.BlockSpec((1, 256), lambda i: (0, 0)),
        ],
        out_specs=pl.BlockSpec((1, 1, 128), lambda i: (i, 0, 0)),
        scratch_shapes=[
            pltpu.VMEM((224, 32, 32), F32),
            pltpu.VMEM((114, 28, 256), F32),
            pltpu.VMEM((112, 32, 768), F32),
            pltpu.VMEM((58, 28, 256), F32),
            pltpu.VMEM((56, 32, 768), F32),
        ],
        compiler_params=pltpu.CompilerParams(
            dimension_semantics=("arbitrary",),
            vmem_limit_bytes=56 * 1024 * 1024,
        ),
        name="cnn_tower",
    )(sdec, w1b, b1b, w2b, b2b, w3b, b3b)

    combined = jnp.concatenate([feat2d[:, 0, :], feat1d], axis=1)
    out = pl.pallas_call(
        _head_kernel,
        out_shape=jax.ShapeDtypeStruct((B, 128), F32),
        name="head_mlp",
    )(combined, wf1, bf1, wf2, bf2)
    return out[:, :1]


# speedup vs baseline: 1.0749x; 1.0493x over previous
import numpy as np
import jax
import jax.numpy as jnp
from jax.experimental import pallas as pl
from jax.experimental.pallas import tpu as pltpu

IMAGE_SIZE = 224
MAX_RADIUS = 112
BN_EPS = 1e-5
NPIX = IMAGE_SIZE * IMAGE_SIZE
F32 = jnp.float32
BF = jnp.bfloat16

_yy, _xx = np.meshgrid(np.arange(IMAGE_SIZE), np.arange(IMAGE_SIZE), indexing="ij")
_r = np.sqrt((_xx - IMAGE_SIZE // 2) ** 2 + (_yy - IMAGE_SIZE // 2) ** 2)
_rint = np.round(_r).astype(np.int32)
_RIDS = np.where(_rint < MAX_RADIUS, _rint, MAX_RADIUS).reshape(-1)
_OH = np.zeros((NPIX, 128), np.float32)
_OH[np.arange(NPIX), _RIDS] = 1.0
_COUNTS = _OH.sum(0)[:MAX_RADIUS]
_CNTR = np.zeros((1, 128), np.float32)
_CNTR[0, :MAX_RADIUS] = 1.0 / (_COUNTS + 1e-8)

_ks = (np.arange(IMAGE_SIZE) + IMAGE_SIZE // 2) % IMAGE_SIZE
_ang = -2.0 * np.pi * np.outer(_ks, np.arange(IMAGE_SIZE)) / IMAGE_SIZE
_FSR = np.cos(_ang).astype(np.float32)
_FSI = np.sin(_ang).astype(np.float32)
_FSRT = np.ascontiguousarray(_FSR.T)
_FSIT = np.ascontiguousarray(_FSI.T)

_EW = np.zeros((224, 280), np.float32)
for _xb in range(28):
    for _dd in range(10):
        _x = 8 * _xb + _dd - 1
        if 0 <= _x < 224:
            _EW[_x, 10 * _xb + _dd] = 1.0


def _silu(v):
    return v * jax.nn.sigmoid(v)


def _hilo(v):
    h = v.astype(BF)
    return h, (v - h.astype(F32)).astype(BF)


def _fft_kernel(x_ref, fr_ref, fi_ref, frt_ref, fit_ref, ew_ref,
                o_ref, o2_ref):
    nb = x_ref.shape[0]
    x = x_ref[...]
    gray = 0.299 * x[:, 0] + 0.587 * x[:, 1] + 0.114 * x[:, 2]
    g2 = gray.reshape(nb * 224, 224)
    hp = jax.lax.Precision.HIGHEST
    ur = jnp.dot(g2, frt_ref[...], preferred_element_type=F32, precision=hp)
    ui = jnp.dot(g2, fit_ref[...], preferred_element_type=F32, precision=hp)
    fr, fi = fr_ref[...], fi_ref[...]
    ew = ew_ref[...]
    for s in range(nb):
        urs = ur[s * 224:(s + 1) * 224]
        uis = ui[s * 224:(s + 1) * 224]
        vr = (jnp.dot(fr, urs, preferred_element_type=F32, precision=hp)
              - jnp.dot(fi, uis, preferred_element_type=F32, precision=hp))
        vi = (jnp.dot(fr, uis, preferred_element_type=F32, precision=hp)
              + jnp.dot(fi, urs, preferred_element_type=F32, precision=hp))
        lm = jnp.log1p(jnp.sqrt(vr * vr + vi * vi))
        m = jnp.sum(lm, keepdims=True) * (1.0 / NPIX)
        d = lm - m
        var = jnp.sum(d * d, keepdims=True) * (1.0 / (NPIX - 1))
        d = d / (jnp.sqrt(var) + 1e-8)
        o_ref[s] = d
        dh, dl = _hilo(d)
        o2_ref[s, 0:1, :] = jnp.zeros((1, 280), F32)
        o2_ref[s, 225:226, :] = jnp.zeros((1, 280), F32)
        o2_ref[s, 1:225, :] = (jnp.dot(dh, ew, preferred_element_type=F32)
                               + jnp.dot(dl, ew, preferred_element_type=F32))


_KCHUNKS = 7
_KROWS = 224 // _KCHUNKS


def _hist_kernel(spec_ref, oh_ref, cntr_ref, w1_ref, b1_ref, w2_ref, b2_ref,
                 o_ref, acc_ref):
    k = pl.program_id(0)

    @pl.when(k == 0)
    def _():
        acc_ref[...] = jnp.zeros_like(acc_ref)

    sp = spec_ref[...]
    tot = jnp.dot(sp[:, 0, :], oh_ref[0], preferred_element_type=F32)
    for y in range(1, _KROWS):
        tot = tot + jnp.dot(sp[:, y, :], oh_ref[y], preferred_element_type=F32)
    acc_ref[...] += tot

    @pl.when(k == _KCHUNKS - 1)
    def _():
        b = acc_ref.shape[0]
        radial = acc_ref[...] * cntr_ref[...]
        lane = jax.lax.broadcasted_iota(jnp.int32, (b, 128), 1)
        mask = lane < MAX_RADIUS
        mean = jnp.sum(radial, axis=1, keepdims=True) * (1.0 / MAX_RADIUS)
        d = jnp.where(mask, radial - mean, 0.0)
        var = jnp.sum(d * d, axis=1, keepdims=True) * (1.0 / (MAX_RADIUS - 1))
        rn = d / (jnp.sqrt(var) + 1e-8)
        g = jnp.dot(rn, w1_ref[...], preferred_element_type=F32) + b1_ref[...]
        g = _silu(g)
        h = jnp.dot(g, w2_ref[...], preferred_element_type=F32) + b2_ref[...]
        o_ref[...] = _silu(h)


def _cnn_kernel(s_ref, w1_ref, b1_ref, w2_ref, b2_ref, w3_ref, b3_ref,
                o_ref, p1, h1w, p2, h2w, p3):
    @pl.when(pl.program_id(0) == 0)
    def _():
        p1[:, 0:28, 30:32] = jnp.zeros((224, 28, 2), F32)
        p1[:, 28:32, :] = jnp.zeros((224, 4, 32), F32)
        p2[:, 28:32, :] = jnp.zeros((112, 4, 768), F32)
        p3[:, 28:32, :] = jnp.zeros((56, 4, 768), F32)
        h1w[0:1] = jnp.zeros((1, 28, 256), F32)
        h1w[113:114] = jnp.zeros((1, 28, 256), F32)
        h1w[:, :, 160:224] = jnp.zeros((114, 28, 64), F32)
        h2w[0:1] = jnp.zeros((1, 28, 256), F32)
        h2w[57:58] = jnp.zeros((1, 28, 256), F32)

    s = s_ref[0]
    p1[:, 0:28, 0:10] = s[0:224]
    p1[:, 0:28, 10:20] = s[1:225]
    p1[:, 0:28, 20:30] = s[2:226]
    w1v, b1v = w1_ref[...], b1_ref[...]
    for c in range(14):
        pc = p1[16 * c:16 * (c + 1)]
        zc = jnp.dot(pc.reshape(512, 32), w1v,
                     preferred_element_type=F32) + b1v
        v = _silu(zc).reshape(8, 2, 32, 256)
        m = jnp.maximum(v[:, 0], v[:, 1])
        m2 = jnp.maximum(m[:, :, 0:128], m[:, :, 128:256])
        h1w[1 + 8 * c:9 + 8 * c, :, 0:128] = m2[:, 0:28, :]
        h1w[1 + 8 * c:9 + 8 * c, :, 128:160] = jnp.concatenate(
            [m2[:, 1:28, 0:32], jnp.zeros((8, 1, 32), F32)], axis=1)
        h1w[1 + 8 * c:9 + 8 * c, :, 224:256] = jnp.concatenate(
            [jnp.zeros((8, 1, 32), F32), m2[:, 0:27, 96:128]], axis=1)
    for dy in range(3):
        p2[:, 0:28, 256 * dy:256 * (dy + 1)] = h1w[dy:dy + 112]
    w2v, b2v = w2_ref[...], b2_ref[...]
    for c in range(7):
        pc = p2[16 * c:16 * (c + 1)]
        zc = jnp.dot(pc.reshape(512, 768), w2v,
                     preferred_element_type=F32) + b2v
        v = _silu(zc).reshape(8, 2, 32, 256)
        m = jnp.maximum(v[:, 0], v[:, 1])
        m2 = jnp.maximum(m[:, :, 0:128], m[:, :, 128:256])
        h2w[1 + 8 * c:9 + 8 * c, :, 0:128] = m2[:, 0:28, :]
        h2w[1 + 8 * c:9 + 8 * c, :, 128:192] = jnp.concatenate(
            [m2[:, 1:28, 0:64], jnp.zeros((8, 1, 64), F32)], axis=1)
        h2w[1 + 8 * c:9 + 8 * c, :, 192:256] = jnp.concatenate(
            [jnp.zeros((8, 1, 64), F32), m2[:, 0:27, 64:128]], axis=1)
    for dy in range(3):
        p3[:, 0:28, 256 * dy:256 * (dy + 1)] = h2w[dy:dy + 56]
    w3v, b3v = w3_ref[...], b3_ref[...]
    tot = jnp.zeros((1, 256), F32)
    for c in range(4):
        pc = p3[14 * c:14 * (c + 1)]
        zc = jnp.dot(pc.reshape(448, 768), w3v,
                     preferred_element_type=F32) + b3v
        sv = _silu(zc).reshape(14, 32, 256)[:, 0:28, :]
        tot = tot + jnp.sum(jnp.sum(sv, axis=0), axis=0, keepdims=True)
    o_ref[0] = (tot[:, 0:128] + tot[:, 128:256]) * (1.0 / 3136.0)


def _head_kernel(c_ref, w1_ref, b1_ref, w2_ref, b2_ref, o_ref):
    z = jnp.dot(c_ref[...], w1_ref[...], preferred_element_type=F32) + b1_ref[...]
    z = _silu(z)
    o_ref[...] = jnp.dot(z, w2_ref[...], preferred_element_type=F32) + b2_ref[...]


def _bn_fold(bn):
    s = bn["g"] / jnp.sqrt(bn["v"] + BN_EPS)
    t = bn["b"] - bn["m"] * s
    return s, t


_ROFF2 = [224, 0, 32, 64, 96, 128]
_ROFF3 = [192, 0, 64, 128]


def kernel(x, params):
    p = params
    B = x.shape[0]

    s1, t1 = _bn_fold(p["bn1"])
    s2, t2 = _bn_fold(p["bn2"])
    s3, t3 = _bn_fold(p["bn3"])

    w1s = p["c1_w"][:, 0] * s1[:, None, None]
    w1b = jnp.zeros((32, 256), F32)
    for dy in range(3):
        for kx in range(3):
            vec = w1s[:, dy, kx]
            for j in range(8):
                c0 = (j % 2) * 128 + (j // 2) * 32
                w1b = w1b.at[dy * 10 + j + kx, c0:c0 + 32].set(vec)

    w2s = p["c2_w"] * s2[:, None, None, None]
    w2b = jnp.zeros((768, 256), F32)
    for dy in range(3):
        for kx in range(3):
            blk = w2s[:, :, dy, kx].T
            for j2 in range(4):
                r0 = dy * 256 + _ROFF2[j2 + kx]
                c0 = (j2 % 2) * 128 + (j2 // 2) * 64
                w2b = w2b.at[r0:r0 + 32, c0:c0 + 64].set(blk)

    w3s = p["c3_w"] * s3[:, None, None, None]
    w3b = jnp.zeros((768, 256), F32)
    for dy in range(3):
        for kx in range(3):
            blk = w3s[:, :, dy, kx].T
            for j3 in range(2):
                r0 = dy * 256 + _ROFF3[j3 + kx]
                w3b = w3b.at[r0:r0 + 64, j3 * 128:(j3 + 1) * 128].set(blk)

    b1b = jnp.tile(p["c1_b"] * s1 + t1, 8)[None, :]
    b2b = jnp.tile(p["c2_b"] * s2 + t2, 4)[None, :]
    b3b = jnp.tile(p["c3_b"] * s3 + t3, 2)[None, :]

    s1d, t1d = _bn_fold(p["bn1d"])
    w1e = jnp.zeros((128, 64), F32).at[:MAX_RADIUS, :].set(
        p["l1_w"].T * s1d[None, :])
    b1e = (p["l1_b"] * s1d + t1d)[None, :]
    w2e = p["l2_w"].T
    b2e = p["l2_b"][None, :]

    wf1 = p["f1_w"].T
    bf1 = p["f1_b"][None, :]
    wf2 = jnp.zeros((64, 128), F32).at[:, 0].set(p["f2_w"][0])
    bf2 = jnp.zeros((1, 128), F32).at[0, 0].set(p["f2_b"][0])

    fsr, fsi = jnp.asarray(_FSR), jnp.asarray(_FSI)
    fsrt, fsit = jnp.asarray(_FSRT), jnp.asarray(_FSIT)
    ewb = jnp.asarray(_EW).astype(BF)

    nb = 8
    c224 = pl.BlockSpec((224, 224), lambda i: (0, 0))
    spec, sdec2 = pl.pallas_call(
        _fft_kernel,
        out_shape=(jax.ShapeDtypeStruct((B, 224, 224), F32),
                   jax.ShapeDtypeStruct((B, 226, 280), F32)),
        grid=(B // nb,),
        in_specs=[pl.BlockSpec((nb, 3, 224, 224), lambda i: (i, 0, 0, 0)),
                  c224, c224, c224, c224,
                  pl.BlockSpec((224, 280), lambda i: (0, 0))],
        out_specs=[pl.BlockSpec((nb, 224, 224), lambda i: (i, 0, 0)),
                   pl.BlockSpec((nb, 226, 280), lambda i: (i, 0, 0))],
        compiler_params=pltpu.CompilerParams(
            dimension_semantics=("arbitrary",),
            vmem_limit_bytes=52 * 1024 * 1024,
        ),
        name="fft_spectrum",
    )(x, fsr, fsi, fsrt, fsit, ewb)

    feat1d = pl.pallas_call(
        _hist_kernel,
        out_shape=jax.ShapeDtypeStruct((B, 64), F32),
        grid=(_KCHUNKS,),
        in_specs=[
            pl.BlockSpec((B, _KROWS, 224), lambda k: (0, k, 0)),
            pl.BlockSpec((_KROWS, 224, 128), lambda k: (k, 0, 0)),
            pl.BlockSpec((1, 128), lambda k: (0, 0)),
            pl.BlockSpec((128, 64), lambda k: (0, 0)),
            pl.BlockSpec((1, 64), lambda k: (0, 0)),
            pl.BlockSpec((64, 64), lambda k: (0, 0)),
            pl.BlockSpec((1, 64), lambda k: (0, 0)),
        ],
        out_specs=pl.BlockSpec((B, 64), lambda k: (0, 0)),
        scratch_shapes=[pltpu.VMEM((B, 128), F32)],
        compiler_params=pltpu.CompilerParams(
            dimension_semantics=("arbitrary",),
            vmem_limit_bytes=48 * 1024 * 1024,
        ),
        name="radial_hist_mlp",
    )(spec, jnp.asarray(_OH).reshape(224, 224, 128),
      jnp.asarray(_CNTR), w1e, b1e, w2e, b2e)

    sdec = sdec2.reshape(B, 226, 28, 10)
    feat2d = pl.pallas_call(
        _cnn_kernel,
        out_shape=jax.ShapeDtypeStruct((B, 1, 128), F32),
        grid=(B,),
        in_specs=[
            pl.BlockSpec((1, 226, 28, 10), lambda i: (i, 0, 0, 0)),
            pl.BlockSpec((32, 256), lambda i: (0, 0)),
            pl.BlockSpec((1, 256), lambda i: (0, 0)),
            pl.BlockSpec((768, 256), lambda i: (0, 0)),
            pl.BlockSpec((1, 256), lambda i: (0, 0)),
            pl.BlockSpec((768, 256), lambda i: (0, 0)),
            pl.BlockSpec((1, 256), lambda i: (0, 0)),
        ],
        out_specs=pl.BlockSpec((1, 1, 128), lambda i: (i, 0, 0)),
        scratch_shapes=[
            pltpu.VMEM((224, 32, 32), F32),
            pltpu.VMEM((114, 28, 256), F32),
            pltpu.VMEM((112, 32, 768), F32),
            pltpu.VMEM((58, 28, 256), F32),
            pltpu.VMEM((56, 32, 768), F32),
        ],
        compiler_params=pltpu.CompilerParams(
            dimension_semantics=("arbitrary",),
            vmem_limit_bytes=56 * 1024 * 1024,
        ),
        name="cnn_tower",
    )(sdec, w1b, b1b, w2b, b2b, w3b, b3b)

    combined = jnp.concatenate([feat2d[:, 0, :], feat1d], axis=1)
    out = pl.pallas_call(
        _head_kernel,
        out_shape=jax.ShapeDtypeStruct((B, 128), F32),
        name="head_mlp",
    )(combined, wf1, bf1, wf2, bf2)
    return out[:, :1]
